# Initial kernel scaffold; baseline (speedup 1.0000x reference)
import jax
import jax.numpy as jnp
from jax.experimental import pallas as pl


def kernel(x, edge_index):
    raise NotImplementedError("write your pallas kernel here")


# speedup vs baseline: 1.6917x; 1.6917x over previous
import jax
import jax.numpy as jnp
from jax import lax
from jax.experimental import pallas as pl
from jax.experimental.pallas import tpu as pltpu
from jax.experimental.pallas import tpu_sc as plsc

N_NODES = 10000
D = 128
NC = 2
NS = 16
NW = NC * NS
R = 320
N_PAD = NW * R
C = 10000
L = 16


def _body(src_hbm, dst_hbm, x_hbm, out_hbm,
          acc, dst_c, src_c, sel_src, sel_dst, rows, sem):
    E = src_hbm.shape[0]
    n_chunks = E // C
    cid = lax.axis_index("c")
    sid = lax.axis_index("s")
    wid = sid * NC + cid
    lo = wid * R
    minus_inf = jnp.full((L,), -jnp.inf, jnp.float32)

    def init_row(r, _):
        for k in range(D // L):
            acc[r, pl.ds(k * L, L)] = minus_inf
        return 0
    lax.fori_loop(0, R + 1, init_row, 0)

    def do_chunk(ci, _):
        base = ci * C
        pltpu.sync_copy(dst_hbm.at[pl.ds(base, C)], dst_c)
        pltpu.sync_copy(src_hbm.at[pl.ds(base, C)], src_c)

        def scan16(i, cnt):
            off = i * L
            d = dst_c[pl.ds(off, L)]
            s = src_c[pl.ds(off, L)]
            m = (d >= lo) & (d < lo + R)
            mi = m.astype(jnp.int32)
            incl = plsc.cumsum(mi)
            pos = cnt + incl - mi
            plsc.store_scatter(sel_dst, [pos], d - lo, mask=m)
            plsc.store_scatter(sel_src, [pos], s, mask=m)
            return cnt + incl[L - 1]
        n = lax.fori_loop(0, C // L, scan16, 0)

        sel_src[pl.ds(n, L)] = jnp.full((L,), wid, jnp.int32)
        sel_dst[pl.ds(n, L)] = jnp.full((L,), R, jnp.int32)

        def group(g, _):
            goff = g * L
            idx16 = sel_src[pl.ds(goff, L)]
            pltpu.async_copy(x_hbm.at[idx16], rows, sem).wait()
            dl = sel_dst[pl.ds(goff, L)]
            for lane in range(L):
                dr = dl[lane]
                for k in range(D // L):
                    sl = pl.ds(k * L, L)
                    acc[dr, sl] = jnp.maximum(acc[dr, sl], rows[lane, sl])
            return 0
        lax.fori_loop(0, (n + L - 1) // L, group, 0)
        return 0
    lax.fori_loop(0, n_chunks, do_chunk, 0)

    zeros = jnp.zeros((L,), jnp.float32)
    def fix_row(r, _):
        for k in range(D // L):
            sl = pl.ds(k * L, L)
            v = acc[r, sl]
            acc[r, sl] = jnp.where(v == -jnp.inf, zeros, v)
        return 0
    lax.fori_loop(0, R, fix_row, 0)
    pltpu.sync_copy(acc.at[pl.ds(0, R)], out_hbm.at[pl.ds(lo, R)])


def kernel(x, edge_index):
    edge_index = edge_index.astype(jnp.int32)
    src = edge_index[0]
    dst = edge_index[1]
    mesh = plsc.VectorSubcoreMesh(
        core_axis_name="c", subcore_axis_name="s",
        num_cores=NC, num_subcores=NS)
    f = pl.kernel(
        _body,
        out_type=jax.ShapeDtypeStruct((N_PAD, D), jnp.float32),
        mesh=mesh,
        compiler_params=pltpu.CompilerParams(needs_layout_passes=False),
        scratch_types=[
            pltpu.VMEM((R + 1, D), jnp.float32),
            pltpu.VMEM((C,), jnp.int32),
            pltpu.VMEM((C,), jnp.int32),
            pltpu.VMEM((C + L,), jnp.int32),
            pltpu.VMEM((C + L,), jnp.int32),
            pltpu.VMEM((L, D), jnp.float32),
            pltpu.SemaphoreType.DMA,
        ],
    )
    out = f(src, dst, x)
    return out[:N_NODES]


# speedup vs baseline: 2.4110x; 1.4252x over previous
import jax
import jax.numpy as jnp
from jax import lax
from jax.experimental import pallas as pl
from jax.experimental.pallas import tpu as pltpu
from jax.experimental.pallas import tpu_sc as plsc

N_NODES = 10000
D = 128
NC = 2
NS = 16
NW = NC * NS
R = 320
N_PAD = NW * R
C = 10000
L = 16
W = 128
SHIFT = 14


def _body(ep_hbm, x_hbm, out_hbm,
          acc, ec, sel_s, sel_d, rows0, rows1, sem0, sem1):
    E = ep_hbm.shape[0]
    n_chunks = E // C
    cid = lax.axis_index("c")
    sid = lax.axis_index("s")
    wid = sid * NC + cid
    lo = wid * R
    minus_inf = jnp.full((L,), -jnp.inf, jnp.float32)

    def init_row(r, _):
        for k in range(D // L):
            acc[r, pl.ds(k * L, L)] = minus_inf
        return 0
    lax.fori_loop(0, R + 1, init_row, 0)

    lob = lo << SHIFT
    hib = (lo + R) << SHIFT
    bufs = ((rows0, sem0), (rows1, sem1))

    def fire(w, buf, sem):
        pltpu.async_copy(x_hbm.at[sel_s.at[pl.ds(w * W, W)]], buf, sem)

    def do_chunk(ci, _):
        base = ci * C
        pltpu.sync_copy(ep_hbm.at[pl.ds(base, C)], ec)

        def scan16(i, cnt_vec):
            p = ec[pl.ds(i * L, L)]
            m = (p >= lob) & (p < hib)
            mi = m.astype(jnp.int32)
            incl = plsc.cumsum(mi)
            pos = cnt_vec + (incl - mi)
            plsc.store_scatter(sel_s, [pos], p & ((1 << SHIFT) - 1), mask=m)
            plsc.store_scatter(sel_d, [pos], (p >> SHIFT) - lo, mask=m)
            return cnt_vec + plsc.all_reduce_population_count(m)
        cnt_vec = lax.fori_loop(0, C // L, scan16, jnp.zeros((L,), jnp.int32))
        n = cnt_vec[0]

        padv = jnp.full((L,), wid, jnp.int32)
        padd = jnp.full((L,), R, jnp.int32)
        for j in range(W // L):
            sel_s[pl.ds(n + j * L, L)] = padv
            sel_d[pl.ds(n + j * L, L)] = padd

        ng = (n + L - 1) // L
        nw = (ng + 7) // 8

        @pl.when(nw > 0)
        def _():
            fire(0, rows0, sem0)
        @pl.when(nw > 1)
        def _():
            fire(1, rows1, sem1)

        def pair(wp, _):
            for b in range(2):
                rows, sem = bufs[b]
                w = wp * 2 + b

                @pl.when(w < nw)
                def _():
                    pltpu.make_async_copy(
                        x_hbm.at[sel_s.at[pl.ds(w * W, W)]], rows, sem).wait()
                    gend = jnp.minimum(8, ng - 8 * w)

                    def grp(j, _):
                        goff = w * W + j * L
                        dl = sel_d[pl.ds(goff, L)]
                        for lane in range(L):
                            dr = dl[lane]
                            rr = j * L + lane
                            for k in range(D // L):
                                sl = pl.ds(k * L, L)
                                acc[dr, sl] = jnp.maximum(acc[dr, sl],
                                                          rows[rr, sl])
                        return 0
                    lax.fori_loop(0, gend, grp, 0)

                    @pl.when(w + 2 < nw)
                    def _():
                        fire(w + 2, rows, sem)
            return 0
        lax.fori_loop(0, (nw + 1) // 2, pair, 0)
        return 0
    lax.fori_loop(0, n_chunks, do_chunk, 0)

    zeros = jnp.zeros((L,), jnp.float32)
    def fix_row(r, _):
        for k in range(D // L):
            sl = pl.ds(k * L, L)
            v = acc[r, sl]
            acc[r, sl] = jnp.where(v == -jnp.inf, zeros, v)
        return 0
    lax.fori_loop(0, R, fix_row, 0)
    pltpu.sync_copy(acc.at[pl.ds(0, R)], out_hbm.at[pl.ds(lo, R)])


def kernel(x, edge_index):
    edge_index = edge_index.astype(jnp.int32)
    ep = edge_index[0] | (edge_index[1] << SHIFT)
    mesh = plsc.VectorSubcoreMesh(
        core_axis_name="c", subcore_axis_name="s",
        num_cores=NC, num_subcores=NS)
    f = pl.kernel(
        _body,
        out_type=jax.ShapeDtypeStruct((N_PAD, D), jnp.float32),
        mesh=mesh,
        compiler_params=pltpu.CompilerParams(needs_layout_passes=False),
        scratch_types=[
            pltpu.VMEM((R + 1, D), jnp.float32),
            pltpu.VMEM((C,), jnp.int32),
            pltpu.VMEM((C + W,), jnp.int32),
            pltpu.VMEM((C + W,), jnp.int32),
            pltpu.VMEM((W, D), jnp.float32),
            pltpu.VMEM((W, D), jnp.float32),
            pltpu.SemaphoreType.DMA,
            pltpu.SemaphoreType.DMA,
        ],
    )
    out = f(ep, x)
    return out[:N_NODES]
